# Initial kernel scaffold; baseline (speedup 1.0000x reference)
import jax
import jax.numpy as jnp
from jax.experimental import pallas as pl


def kernel(x, y):
    raise NotImplementedError("write your pallas kernel here")


# speedup vs baseline: 13.4541x; 13.4541x over previous
import jax
import jax.numpy as jnp
from jax.experimental import pallas as pl
from jax.experimental.pallas import tpu as pltpu

_N = 8192
_TILE = 512
_GRID = _N // _TILE
_INF = float("inf")


def _chamfer_body(xp_ref, ytp_ref, out_ref, colmin_ref, colex_ref, rowsum_ref):
    i = pl.program_id(0)

    x = xp_ref[...]
    yt = ytp_ref[...]

    x2 = jnp.sum(x * x, axis=1, keepdims=True)
    y2 = jnp.sum(yt * yt, axis=0, keepdims=True)
    p = jnp.dot(x.astype(jnp.bfloat16), yt.astype(jnp.bfloat16),
                preferred_element_type=jnp.float32)
    d2a = (x2 - 2.0 * p) + y2

    d0 = x[:, 0:1] - yt[0:1, :]
    d1 = x[:, 1:2] - yt[1:2, :]
    d2 = x[:, 2:3] - yt[2:3, :]
    d2e = d0 * d0 + d1 * d1 + d2 * d2

    rmin_a = jnp.min(d2a, axis=1, keepdims=True)
    rex = jnp.min(jnp.where(d2a == rmin_a, d2e, _INF), axis=1)
    rsum = jnp.sum(rex)

    cmin_t = jnp.min(d2a, axis=0, keepdims=True)
    cex_t = jnp.min(jnp.where(d2a == cmin_t, d2e, _INF), axis=0, keepdims=True)

    @pl.when(i == 0)
    def _init():
        rowsum_ref[0] = rsum
        colmin_ref[...] = cmin_t
        colex_ref[...] = cex_t

    @pl.when(i > 0)
    def _acc():
        rowsum_ref[0] = rowsum_ref[0] + rsum
        upd = cmin_t < colmin_ref[...]
        colex_ref[...] = jnp.where(upd, cex_t, colex_ref[...])
        colmin_ref[...] = jnp.minimum(colmin_ref[...], cmin_t)

    @pl.when(i == _GRID - 1)
    def _fin():
        total = rowsum_ref[0] + jnp.sum(colex_ref[...])
        out_ref[0, 0] = total * (1.0 / (_N * 3.0))


@jax.jit
def kernel(x, y):
    xp = jnp.concatenate([x, jnp.zeros((_N, 5), jnp.float32)], axis=1)
    ytp = jnp.concatenate([y.T, jnp.zeros((5, _N), jnp.float32)], axis=0)

    out = pl.pallas_call(
        _chamfer_body,
        grid=(_GRID,),
        in_specs=[
            pl.BlockSpec((_TILE, 8), lambda i: (i, 0)),
            pl.BlockSpec((8, _N), lambda i: (0, 0)),
        ],
        out_specs=pl.BlockSpec((1, 1), lambda i: (0, 0), memory_space=pltpu.SMEM),
        out_shape=jax.ShapeDtypeStruct((1, 1), jnp.float32),
        scratch_shapes=[
            pltpu.VMEM((1, _N), jnp.float32),
            pltpu.VMEM((1, _N), jnp.float32),
            pltpu.SMEM((1,), jnp.float32),
        ],
    )(xp, ytp)
    return out[0, 0]


# speedup vs baseline: 18.6035x; 1.3827x over previous
import functools

import jax
import jax.numpy as jnp
from jax import lax
from jax.experimental import pallas as pl
from jax.experimental.pallas import tpu as pltpu
from jax.experimental.pallas import tpu_sc as plsc

_N = 8192
_TILE = 512
_GRID = _N // _TILE

_NC = 2
_NS = 16
_NW = _NC * _NS
_CHUNK = _N // _NW
_G = 128
_D = 16
_GD = 128


def _argmin_body(xp_ref, ytp_ref, ridx_ref, cidx_ref,
                 colmin_ref, colidx_ref):
    i = pl.program_id(0)

    x = xp_ref[...]
    yt = ytp_ref[...]

    x2 = jnp.sum(x * x, axis=1, keepdims=True)
    y2 = jnp.sum(yt * yt, axis=0, keepdims=True)
    p = jnp.dot(x.astype(jnp.bfloat16), yt.astype(jnp.bfloat16),
                preferred_element_type=jnp.float32)
    d2a = (x2 - 2.0 * p) + y2

    lane_iota = lax.broadcasted_iota(jnp.int32, (_TILE, _N), 1)
    sub_iota = lax.broadcasted_iota(jnp.int32, (_TILE, _N), 0) + i * _TILE

    rmin = jnp.min(d2a, axis=1, keepdims=True)
    ridx_ref[...] = jnp.min(jnp.where(d2a == rmin, lane_iota, _N),
                            axis=1, keepdims=True)

    cmin_t = jnp.min(d2a, axis=0, keepdims=True)
    cidx_t = jnp.min(jnp.where(d2a == cmin_t, sub_iota, _N),
                     axis=0, keepdims=True)

    @pl.when(i == 0)
    def _init():
        colmin_ref[...] = cmin_t
        colidx_ref[...] = cidx_t

    @pl.when(i > 0)
    def _acc():
        upd = cmin_t < colmin_ref[...]
        colidx_ref[...] = jnp.where(upd, cidx_t, colidx_ref[...])
        colmin_ref[...] = jnp.minimum(colmin_ref[...], cmin_t)

    @pl.when(i == _GRID - 1)
    def _fin():
        cidx_ref[...] = colidx_ref[...]


def _phase1(xp, ytp):
    return pl.pallas_call(
        _argmin_body,
        grid=(_GRID,),
        in_specs=[
            pl.BlockSpec((_TILE, 8), lambda i: (i, 0)),
            pl.BlockSpec((8, _N), lambda i: (0, 0)),
        ],
        out_specs=[
            pl.BlockSpec((_TILE, 1), lambda i: (i, 0)),
            pl.BlockSpec((1, _N), lambda i: (0, 0)),
        ],
        out_shape=[
            jax.ShapeDtypeStruct((_N, 1), jnp.int32),
            jax.ShapeDtypeStruct((1, _N), jnp.int32),
        ],
        scratch_shapes=[
            pltpu.VMEM((1, _N), jnp.float32),
            pltpu.VMEM((1, _N), jnp.int32),
        ],
    )(xp, ytp)


def _sc_mse_body(xq_hbm, yq_hbm, xg_hbm, yg_hbm, ridx_hbm, cidx_hbm, out_hbm,
                 idx_v, rows_v, q_v, acc_v, shared, res_v, sem):
    cid = lax.axis_index("c")
    sid = lax.axis_index("s")
    wid = cid * _NS + sid
    base = wid * _CHUNK
    ibase = wid * (_CHUNK // _G)

    acc_v[...] = jnp.zeros((_D,), jnp.float32)

    for q_hbm, t_hbm, i_hbm in ((xq_hbm, yg_hbm, ridx_hbm),
                                (yq_hbm, xg_hbm, cidx_hbm)):
        pltpu.sync_copy(i_hbm.at[pl.ds(ibase, _CHUNK // _G)], idx_v)
        pltpu.sync_copy(q_hbm.at[pl.ds(base, _CHUNK)], q_v)
        for j in range(_CHUNK // _G):
            pltpu.async_copy(t_hbm.at[idx_v.at[j]],
                             rows_v.at[pl.ds(j * _G, _G)], sem).wait()

        def body(r, acc):
            d = q_v[r, :] - rows_v[r, 0:_D]
            return acc + d * d

        acc_v[...] = lax.fori_loop(0, _CHUNK, body, acc_v[...])

    pltpu.sync_copy(acc_v, shared.at[sid])
    plsc.subcore_barrier()

    @pl.when(sid == 0)
    def _reduce():
        tot = jnp.zeros((_D,), jnp.float32)
        for w in range(_NS):
            pltpu.sync_copy(shared.at[w], acc_v)
            tot = tot + acc_v[...]
        res_v[...] = tot * jnp.float32(1.0 / (_N * 3.0))
        pltpu.sync_copy(res_v, out_hbm.at[cid])


def _phase2(xq, yq, xg, yg, ridx, cidx):
    mesh = plsc.VectorSubcoreMesh(core_axis_name="c", subcore_axis_name="s")
    kern = functools.partial(
        pl.kernel,
        mesh=mesh,
        out_type=jax.ShapeDtypeStruct((_NC, _D), jnp.float32),
        scratch_types=[
            pltpu.VMEM((_CHUNK // _G, _G), jnp.int32),
            pltpu.VMEM((_CHUNK, _GD), jnp.float32),
            pltpu.VMEM((_CHUNK, _D), jnp.float32),
            pltpu.VMEM((_D,), jnp.float32),
            pltpu.VMEM_SHARED((_NS, _D), jnp.float32),
            pltpu.VMEM((_D,), jnp.float32),
            pltpu.SemaphoreType.DMA,
        ],
    )(_sc_mse_body)
    return kern(xq, yq, xg, yg, ridx, cidx)


@jax.jit
def kernel(x, y):
    xp = jnp.concatenate([x, jnp.zeros((_N, 5), jnp.float32)], axis=1)
    ytp = jnp.concatenate([y.T, jnp.zeros((5, _N), jnp.float32)], axis=0)
    ridx, cidx = _phase1(xp, ytp)

    xq = jnp.concatenate([x, jnp.zeros((_N, _D - 3), jnp.float32)], axis=1)
    yq = jnp.concatenate([y, jnp.zeros((_N, _D - 3), jnp.float32)], axis=1)
    xg = jnp.concatenate([x, jnp.zeros((_N, _GD - 3), jnp.float32)], axis=1)
    yg = jnp.concatenate([y, jnp.zeros((_N, _GD - 3), jnp.float32)], axis=1)
    out = _phase2(xq, yq, xg, yg,
                  ridx.reshape(_N // _G, _G), cidx.reshape(_N // _G, _G))
    return jnp.sum(out)


# speedup vs baseline: 25.1029x; 1.3494x over previous
import functools

import jax
import jax.numpy as jnp
from jax import lax
from jax.experimental import pallas as pl
from jax.experimental.pallas import tpu as pltpu
from jax.experimental.pallas import tpu_sc as plsc

_N = 8192
_TILE = 512
_GRID = _N // _TILE

_NC = 2
_NS = 16
_NW = _NC * _NS
_CHUNK = _N // _NW
_G = 128
_D = 16
_GD = 128


def _argmin_body(xp_ref, ytp_ref, ridx_ref, cidx_ref, colpk_ref):
    i = pl.program_id(0)

    xm2 = xp_ref[...]
    yt = ytp_ref[...]

    x2 = 0.25 * jnp.sum(xm2 * xm2, axis=1, keepdims=True)
    y2 = jnp.sum(yt * yt, axis=0, keepdims=True)
    p = jnp.dot(xm2.astype(jnp.bfloat16), yt.astype(jnp.bfloat16),
                preferred_element_type=jnp.float32)
    d2a = (x2 + p) + y2

    bits = lax.bitcast_convert_type(jnp.maximum(d2a, 1e-30),
                                    jnp.int32) & ~0x1FFF
    lane_iota = lax.broadcasted_iota(jnp.int32, (_TILE, _N), 1)
    grow_iota = lax.broadcasted_iota(jnp.int32, (_TILE, _N), 0) + i * _TILE

    rowpk = jnp.min(lax.bitcast_convert_type(bits | lane_iota, jnp.float32),
                    axis=1, keepdims=True)
    ridx_ref[...] = lax.bitcast_convert_type(rowpk, jnp.int32) & 0x1FFF

    colpk_t = jnp.min(lax.bitcast_convert_type(bits | grow_iota, jnp.float32),
                      axis=0, keepdims=True)

    @pl.when(i == 0)
    def _init():
        colpk_ref[...] = colpk_t

    @pl.when(i > 0)
    def _acc():
        colpk_ref[...] = jnp.minimum(colpk_ref[...], colpk_t)

    @pl.when(i == _GRID - 1)
    def _fin():
        cidx_ref[...] = lax.bitcast_convert_type(colpk_ref[...],
                                                 jnp.int32) & 0x1FFF


def _phase1(xp, ytp):
    return pl.pallas_call(
        _argmin_body,
        grid=(_GRID,),
        in_specs=[
            pl.BlockSpec((_TILE, 8), lambda i: (i, 0)),
            pl.BlockSpec((8, _N), lambda i: (0, 0)),
        ],
        out_specs=[
            pl.BlockSpec((_TILE, 1), lambda i: (i, 0)),
            pl.BlockSpec((1, _N), lambda i: (0, 0)),
        ],
        out_shape=[
            jax.ShapeDtypeStruct((_N, 1), jnp.int32),
            jax.ShapeDtypeStruct((1, _N), jnp.int32),
        ],
        scratch_shapes=[
            pltpu.VMEM((1, _N), jnp.float32),
        ],
    )(xp, ytp)


def _sc_mse_body(xq_hbm, yq_hbm, xg_hbm, yg_hbm, ridx_hbm, cidx_hbm, out_hbm,
                 idx_v, rows_v, q_v, acc_v, shared, res_v, sem):
    cid = lax.axis_index("c")
    sid = lax.axis_index("s")
    wid = cid * _NS + sid
    base = wid * _CHUNK
    ibase = wid * (_CHUNK // _G)

    acc_v[...] = jnp.zeros((_D,), jnp.float32)

    for q_hbm, t_hbm, i_hbm in ((xq_hbm, yg_hbm, ridx_hbm),
                                (yq_hbm, xg_hbm, cidx_hbm)):
        pltpu.sync_copy(i_hbm.at[pl.ds(ibase, _CHUNK // _G)], idx_v)
        pltpu.sync_copy(q_hbm.at[pl.ds(base, _CHUNK)], q_v)
        for j in range(_CHUNK // _G):
            pltpu.async_copy(t_hbm.at[idx_v.at[j]],
                             rows_v.at[pl.ds(j * _G, _G)], sem).wait()

        def body(r, acc):
            d = q_v[r, :] - rows_v[r, 0:_D]
            return acc + d * d

        acc_v[...] = lax.fori_loop(0, _CHUNK, body, acc_v[...])

    pltpu.sync_copy(acc_v, shared.at[sid])
    plsc.subcore_barrier()

    @pl.when(sid == 0)
    def _reduce():
        tot = jnp.zeros((_D,), jnp.float32)
        for w in range(_NS):
            pltpu.sync_copy(shared.at[w], acc_v)
            tot = tot + acc_v[...]
        res_v[...] = tot * jnp.float32(1.0 / (_N * 3.0))
        pltpu.sync_copy(res_v, out_hbm.at[cid])


def _phase2(xq, yq, xg, yg, ridx, cidx):
    mesh = plsc.VectorSubcoreMesh(core_axis_name="c", subcore_axis_name="s")
    kern = functools.partial(
        pl.kernel,
        mesh=mesh,
        out_type=jax.ShapeDtypeStruct((_NC, _D), jnp.float32),
        scratch_types=[
            pltpu.VMEM((_CHUNK // _G, _G), jnp.int32),
            pltpu.VMEM((_CHUNK, _GD), jnp.float32),
            pltpu.VMEM((_CHUNK, _D), jnp.float32),
            pltpu.VMEM((_D,), jnp.float32),
            pltpu.VMEM_SHARED((_NS, _D), jnp.float32),
            pltpu.VMEM((_D,), jnp.float32),
            pltpu.SemaphoreType.DMA,
        ],
    )(_sc_mse_body)
    return kern(xq, yq, xg, yg, ridx, cidx)


@jax.jit
def kernel(x, y):
    xp = jnp.concatenate([-2.0 * x, jnp.zeros((_N, 5), jnp.float32)], axis=1)
    ytp = jnp.concatenate([y.T, jnp.zeros((5, _N), jnp.float32)], axis=0)
    ridx, cidx = _phase1(xp, ytp)

    xq = jnp.concatenate([x, jnp.zeros((_N, _D - 3), jnp.float32)], axis=1)
    yq = jnp.concatenate([y, jnp.zeros((_N, _D - 3), jnp.float32)], axis=1)
    xg = jnp.concatenate([x, jnp.zeros((_N, _GD - 3), jnp.float32)], axis=1)
    yg = jnp.concatenate([y, jnp.zeros((_N, _GD - 3), jnp.float32)], axis=1)
    out = _phase2(xq, yq, xg, yg,
                  ridx.reshape(_N // _G, _G), cidx.reshape(_N // _G, _G))
    return jnp.sum(out)


# speedup vs baseline: 25.1546x; 1.0021x over previous
import functools

import jax
import jax.numpy as jnp
from jax import lax
from jax.experimental import pallas as pl
from jax.experimental.pallas import tpu as pltpu
from jax.experimental.pallas import tpu_sc as plsc

_N = 8192
_TILE = 512
_GRID = _N // _TILE

_NC = 2
_NS = 16
_NW = _NC * _NS
_CHUNK = _N // _NW
_G = 128
_D = 16
_GD = 128


def _argmin_body(xp_ref, ytp_ref, ridx_ref, cidx_ref, colpk_ref, rowid_ref):
    i = pl.program_id(0)

    xm2 = xp_ref[:, 0:8]
    yt = ytp_ref[...]

    x2 = 0.25 * jnp.sum(xm2 * xm2, axis=1, keepdims=True)
    y2 = jnp.sum(yt * yt, axis=0, keepdims=True)
    p = jnp.dot(xm2.astype(jnp.bfloat16), yt.astype(jnp.bfloat16),
                preferred_element_type=jnp.float32)
    d2a = (x2 + p) + y2

    bits = lax.bitcast_convert_type(jnp.maximum(d2a, 1e-30),
                                    jnp.int32) & ~0x1FFF
    lane_iota = lax.broadcasted_iota(jnp.int32, (_TILE, _N), 1)
    grow_iota = lax.broadcasted_iota(jnp.int32, (_TILE, _N), 0) + i * _TILE

    rowpk = jnp.min(lax.bitcast_convert_type(bits | lane_iota, jnp.float32),
                    axis=1, keepdims=True)
    rowid = lax.bitcast_convert_type(rowpk, jnp.int32) & 0x1FFF
    rowid_ref[0:1, pl.ds(i * _TILE, _TILE)] = jnp.reshape(rowid, (1, _TILE))

    colpk_t = jnp.min(lax.bitcast_convert_type(bits | grow_iota, jnp.float32),
                      axis=0, keepdims=True)

    @pl.when(i == 0)
    def _init():
        colpk_ref[...] = colpk_t

    @pl.when(i > 0)
    def _acc():
        colpk_ref[...] = jnp.minimum(colpk_ref[...], colpk_t)

    @pl.when(i == _GRID - 1)
    def _fin():
        ridx_ref[...] = jnp.reshape(rowid_ref[...], (_N // 128, 128))
        cidx_ref[...] = jnp.reshape(
            lax.bitcast_convert_type(colpk_ref[...], jnp.int32) & 0x1FFF,
            (_N // 128, 128))


def _phase1(xp, ytp):
    return pl.pallas_call(
        _argmin_body,
        grid=(_GRID,),
        in_specs=[
            pl.BlockSpec((_TILE, _GD), lambda i: (i, 0)),
            pl.BlockSpec((8, _N), lambda i: (0, 0)),
        ],
        out_specs=[
            pl.BlockSpec((_N // 128, 128), lambda i: (0, 0)),
            pl.BlockSpec((_N // 128, 128), lambda i: (0, 0)),
        ],
        out_shape=[
            jax.ShapeDtypeStruct((_N // 128, 128), jnp.int32),
            jax.ShapeDtypeStruct((_N // 128, 128), jnp.int32),
        ],
        scratch_shapes=[
            pltpu.VMEM((1, _N), jnp.float32),
            pltpu.VMEM((1, _N), jnp.int32),
        ],
    )(xp, ytp)


def _sc_mse_body(xg_hbm, yg_hbm, ridx_hbm, cidx_hbm, out_hbm,
                 idx_v, rows_v, q_v, acc_v, shared, res_v, sem):
    cid = lax.axis_index("c")
    sid = lax.axis_index("s")
    wid = cid * _NS + sid
    base = wid * _CHUNK
    ibase = wid * (_CHUNK // _G)

    acc_v[...] = jnp.zeros((_D,), jnp.float32)

    for q_hbm, t_hbm, i_hbm in ((xg_hbm, yg_hbm, ridx_hbm),
                                (yg_hbm, xg_hbm, cidx_hbm)):
        pltpu.sync_copy(i_hbm.at[pl.ds(ibase, _CHUNK // _G)], idx_v)
        pltpu.sync_copy(q_hbm.at[pl.ds(base, _CHUNK)], q_v)
        for j in range(_CHUNK // _G):
            pltpu.async_copy(t_hbm.at[idx_v.at[j]],
                             rows_v.at[pl.ds(j * _G, _G)], sem).wait()

        def body(r, acc):
            d = q_v[r, 0:_D] - rows_v[r, 0:_D]
            return acc + d * d

        acc_v[...] = lax.fori_loop(0, _CHUNK, body, acc_v[...])

    pltpu.sync_copy(acc_v, shared.at[sid])
    plsc.subcore_barrier()

    @pl.when(sid == 0)
    def _reduce():
        tot = jnp.zeros((_D,), jnp.float32)
        for w in range(_NS):
            pltpu.sync_copy(shared.at[w], acc_v)
            tot = tot + acc_v[...]
        res_v[...] = tot * jnp.float32(0.25 / (_N * 3.0))
        pltpu.sync_copy(res_v, out_hbm.at[cid])


def _phase2(xg, yg, ridx, cidx):
    mesh = plsc.VectorSubcoreMesh(core_axis_name="c", subcore_axis_name="s")
    kern = functools.partial(
        pl.kernel,
        mesh=mesh,
        out_type=jax.ShapeDtypeStruct((_NC, _D), jnp.float32),
        scratch_types=[
            pltpu.VMEM((_CHUNK // _G, _G), jnp.int32),
            pltpu.VMEM((_CHUNK, _GD), jnp.float32),
            pltpu.VMEM((_CHUNK, _GD), jnp.float32),
            pltpu.VMEM((_D,), jnp.float32),
            pltpu.VMEM_SHARED((_NS, _D), jnp.float32),
            pltpu.VMEM((_D,), jnp.float32),
            pltpu.SemaphoreType.DMA,
        ],
    )(_sc_mse_body)
    return kern(xg, yg, ridx, cidx)


@jax.jit
def kernel(x, y):
    xg = jnp.concatenate([-2.0 * x, jnp.zeros((_N, _GD - 3), jnp.float32)],
                         axis=1)
    yg = jnp.concatenate([-2.0 * y, jnp.zeros((_N, _GD - 3), jnp.float32)],
                         axis=1)
    ytp = jnp.concatenate([y.T, jnp.zeros((5, _N), jnp.float32)], axis=0)
    ridx, cidx = _phase1(xg, ytp)
    out = _phase2(xg, yg, ridx, cidx)
    return jnp.sum(out)
